# baseline (speedup 1.0000x reference)
import functools

import jax
import jax.numpy as jnp
from jax.experimental import pallas as pl
from jax.experimental.pallas import tpu as pltpu

B = 128
T = 20
EMB = 128
NA = 100000
TILE = 4096
N_TILES = (NA + TILE - 1) // TILE


def _body(states_ref, w1_ref, b1_ref, w2_ref, b2_ref, out_ref,
          h_ref, logits_ref, m_ref, s_ref):
    i = pl.program_id(0)

    @pl.when(i == 0)
    def _init():
        g_hat = jnp.mean(states_ref[:, :T, :], axis=1)
        x = states_ref[:, T, :]
        hcat = jnp.concatenate([g_hat, x], axis=1)
        pre = jnp.dot(hcat, w1_ref[...],
                      preferred_element_type=jnp.float32) + b1_ref[...]
        h_ref[...] = jnp.maximum(pre, 0.0)
        m_ref[...] = jnp.full_like(m_ref, -jnp.inf)
        s_ref[...] = jnp.zeros_like(s_ref)

    @pl.when(i < N_TILES)
    def _compute():
        logits = jnp.dot(h_ref[...], w2_ref[...],
                         preferred_element_type=jnp.float32) + b2_ref[...]
        col = i * TILE + jax.lax.broadcasted_iota(jnp.int32, (1, TILE), 1)
        logits = jnp.where(col < NA, logits, -jnp.inf)
        logits_ref[:, pl.ds(i * TILE, TILE)] = logits
        tmax = jnp.max(logits, axis=1, keepdims=True)
        m_new = jnp.maximum(m_ref[...], tmax)
        s_ref[...] = (s_ref[...] * jnp.exp(m_ref[...] - m_new)
                      + jnp.sum(jnp.exp(logits - m_new), axis=1,
                                keepdims=True))
        m_ref[...] = m_new

    @pl.when(i >= N_TILES)
    def _normalize():
        j = i - N_TILES
        l = logits_ref[:, pl.ds(j * TILE, TILE)]
        out_ref[...] = jnp.exp(l - m_ref[...]) / s_ref[...]


@functools.partial(jax.jit, static_argnames=())
def kernel(states, W1, b1, W2, b2):
    b1r = b1.reshape(1, EMB)
    b2r = b2.reshape(1, NA)
    grid = (2 * N_TILES,)
    out = pl.pallas_call(
        _body,
        grid=grid,
        in_specs=[
            pl.BlockSpec((B, T + 1, EMB), lambda i: (0, 0, 0)),
            pl.BlockSpec((2 * EMB, EMB), lambda i: (0, 0)),
            pl.BlockSpec((1, EMB), lambda i: (0, 0)),
            pl.BlockSpec((EMB, TILE),
                         lambda i: (0, jnp.minimum(i, N_TILES - 1))),
            pl.BlockSpec((1, TILE),
                         lambda i: (0, jnp.minimum(i, N_TILES - 1))),
        ],
        out_specs=pl.BlockSpec((B, TILE),
                               lambda i: (0, jnp.maximum(i - N_TILES, 0))),
        out_shape=jax.ShapeDtypeStruct((B, NA), jnp.float32),
        scratch_shapes=[
            pltpu.VMEM((B, EMB), jnp.float32),
            pltpu.VMEM((B, N_TILES * TILE), jnp.float32),
            pltpu.VMEM((B, 1), jnp.float32),
            pltpu.VMEM((B, 1), jnp.float32),
        ],
        compiler_params=pltpu.CompilerParams(
            dimension_semantics=("arbitrary",),
            vmem_limit_bytes=120 * 1024 * 1024,
        ),
    )(states, W1, b1r, W2, b2r)
    return out


# speedup vs baseline: 1.0011x; 1.0011x over previous
import functools

import jax
import jax.numpy as jnp
from jax.experimental import pallas as pl
from jax.experimental.pallas import tpu as pltpu

B = 128
T = 20
EMB = 128
NA = 100000
TILE = 4096
N_TILES = (NA + TILE - 1) // TILE


def _body(states_ref, w1_ref, b1_ref, w2_ref, b2_ref, out_ref,
          h_ref, logits_ref, m_ref, s_ref):
    i = pl.program_id(0)

    @pl.when(i == 0)
    def _init():
        g_hat = jnp.mean(states_ref[:, :T, :], axis=1)
        x = states_ref[:, T, :]
        hcat = jnp.concatenate([g_hat, x], axis=1)
        pre = jnp.dot(hcat, w1_ref[...],
                      preferred_element_type=jnp.float32) + b1_ref[...]
        h_ref[...] = jnp.maximum(pre, 0.0).astype(jnp.bfloat16)
        m_ref[...] = jnp.full_like(m_ref, -jnp.inf)
        s_ref[...] = jnp.zeros_like(s_ref)

    @pl.when(i < N_TILES)
    def _compute():
        logits = jnp.dot(h_ref[...], w2_ref[...].astype(jnp.bfloat16),
                         preferred_element_type=jnp.float32) + b2_ref[...]
        col = i * TILE + jax.lax.broadcasted_iota(jnp.int32, (1, TILE), 1)
        logits = jnp.where(col < NA, logits, -jnp.inf)
        logits_ref[:, pl.ds(i * TILE, TILE)] = logits
        tmax = jnp.max(logits, axis=1, keepdims=True)
        m_new = jnp.maximum(m_ref[...], tmax)
        s_ref[...] = (s_ref[...] * jnp.exp(m_ref[...] - m_new)
                      + jnp.sum(jnp.exp(logits - m_new), axis=1,
                                keepdims=True))
        m_ref[...] = m_new

    @pl.when(i >= N_TILES)
    def _normalize():
        j = i - N_TILES
        l = logits_ref[:, pl.ds(j * TILE, TILE)]
        out_ref[...] = jnp.exp(l - m_ref[...]) * (1.0 / s_ref[...])


@functools.partial(jax.jit, static_argnames=())
def kernel(states, W1, b1, W2, b2):
    b1r = b1.reshape(1, EMB)
    b2r = b2.reshape(1, NA)
    grid = (2 * N_TILES,)
    out = pl.pallas_call(
        _body,
        grid=grid,
        in_specs=[
            pl.BlockSpec((B, T + 1, EMB), lambda i: (0, 0, 0)),
            pl.BlockSpec((2 * EMB, EMB), lambda i: (0, 0)),
            pl.BlockSpec((1, EMB), lambda i: (0, 0)),
            pl.BlockSpec((EMB, TILE),
                         lambda i: (0, jnp.minimum(i, N_TILES - 1))),
            pl.BlockSpec((1, TILE),
                         lambda i: (0, jnp.minimum(i, N_TILES - 1))),
        ],
        out_specs=pl.BlockSpec((B, TILE),
                               lambda i: (0, jnp.maximum(i - N_TILES, 0))),
        out_shape=jax.ShapeDtypeStruct((B, NA), jnp.float32),
        scratch_shapes=[
            pltpu.VMEM((B, EMB), jnp.bfloat16),
            pltpu.VMEM((B, N_TILES * TILE), jnp.float32),
            pltpu.VMEM((B, 1), jnp.float32),
            pltpu.VMEM((B, 1), jnp.float32),
        ],
        compiler_params=pltpu.CompilerParams(
            dimension_semantics=("arbitrary",),
            vmem_limit_bytes=120 * 1024 * 1024,
        ),
    )(states, W1, b1r, W2, b2r)
    return out


# speedup vs baseline: 2.4876x; 2.4848x over previous
import functools

import jax
import jax.numpy as jnp
from jax.experimental import pallas as pl
from jax.experimental.pallas import tpu as pltpu

B = 128
T = 20
EMB = 128
NA = 100000
TILE = 4096
N_TILES = (NA + TILE - 1) // TILE


def _body(states_ref, w1_ref, b1_ref, w2t_ref, b2_ref, out_ref,
          h_ref, logits_ref, m_ref, s_ref):
    i = pl.program_id(0)

    @pl.when(i == 0)
    def _init():
        g_hat = jnp.mean(states_ref[:T], axis=0)
        x = states_ref[T]
        hcat = jnp.concatenate([g_hat, x], axis=1)
        pre = jnp.dot(hcat, w1_ref[...],
                      preferred_element_type=jnp.float32) + b1_ref[...]
        h_ref[...] = jnp.maximum(pre, 0.0)
        m_ref[...] = jnp.full_like(m_ref, -jnp.inf)
        s_ref[...] = jnp.zeros_like(s_ref)

    @pl.when(i < N_TILES)
    def _compute():
        b2col = b2_ref[...].reshape(TILE, 1)
        logits = jax.lax.dot_general(
            w2t_ref[...], h_ref[...],
            (((1,), (1,)), ((), ())),
            preferred_element_type=jnp.float32) + b2col
        row = i * TILE + jax.lax.broadcasted_iota(jnp.int32, (TILE, 1), 0)
        logits = jnp.where(row < NA, logits, -jnp.inf)
        logits_ref[pl.ds(i * TILE, TILE), :] = logits
        tmax = jnp.max(logits, axis=0, keepdims=True)
        m_new = jnp.maximum(m_ref[...], tmax)
        s_ref[...] = (s_ref[...] * jnp.exp(m_ref[...] - m_new)
                      + jnp.sum(jnp.exp(logits - m_new), axis=0,
                                keepdims=True))
        m_ref[...] = m_new

    @pl.when(i >= N_TILES)
    def _normalize():
        j = i - N_TILES
        l = logits_ref[pl.ds(j * TILE, TILE), :]
        out_ref[...] = jnp.exp(l - m_ref[...]) * (1.0 / s_ref[...])


@functools.partial(jax.jit, static_argnames=())
def kernel(states, W1, b1, W2, b2):
    states_t = jnp.transpose(states, (1, 0, 2))
    w2t = W2.T
    b1r = b1.reshape(1, EMB)
    b2r = b2.reshape(1, NA)
    grid = (2 * N_TILES,)
    out_t = pl.pallas_call(
        _body,
        grid=grid,
        in_specs=[
            pl.BlockSpec((T + 1, B, EMB), lambda i: (0, 0, 0)),
            pl.BlockSpec((2 * EMB, EMB), lambda i: (0, 0)),
            pl.BlockSpec((1, EMB), lambda i: (0, 0)),
            pl.BlockSpec((TILE, EMB),
                         lambda i: (jnp.minimum(i, N_TILES - 1), 0)),
            pl.BlockSpec((1, TILE),
                         lambda i: (0, jnp.minimum(i, N_TILES - 1))),
        ],
        out_specs=pl.BlockSpec((TILE, B),
                               lambda i: (jnp.maximum(i - N_TILES, 0), 0)),
        out_shape=jax.ShapeDtypeStruct((NA, B), jnp.float32),
        scratch_shapes=[
            pltpu.VMEM((B, EMB), jnp.float32),
            pltpu.VMEM((N_TILES * TILE, B), jnp.float32),
            pltpu.VMEM((1, B), jnp.float32),
            pltpu.VMEM((1, B), jnp.float32),
        ],
        compiler_params=pltpu.CompilerParams(
            dimension_semantics=("arbitrary",),
            vmem_limit_bytes=120 * 1024 * 1024,
        ),
    )(states_t, W1, b1r, w2t, b2r)
    return out_t.T
